# Initial kernel scaffold; baseline (speedup 1.0000x reference)
import jax
import jax.numpy as jnp
from jax.experimental import pallas as pl


def kernel(edge, batch, edge_index, W1, W2, W3, gamma, beta, running_mean, running_var):
    raise NotImplementedError("write your pallas kernel here")


# speedup vs baseline: 11.6391x; 11.6391x over previous
import jax
import jax.numpy as jnp
from jax import lax
from jax.experimental import pallas as pl
from jax.experimental.pallas import tpu as pltpu
from jax.experimental.pallas import tpu_sc as plsc

_E = 320000
_N = 10000
_H = 128
_NG = 512
_EPS = 1e-5

_NC = 2
_NS = 16
_NW = _NC * _NS
_B = 128
_NBLK = _E // _B
_ROUNDS = (_NBLK + _NW - 1) // _NW


def _sc_agg_body(edge_hbm, ei0_hbm, batch_hbm, zeros_hbm, out_hbm,
                 batch_v, ei_v, gid_v, rows_v, acc_sh):
    c = lax.axis_index("c")
    s = lax.axis_index("s")
    wid = s * _NC + c

    pltpu.sync_copy(batch_hbm, batch_v)

    @pl.when(s == 0)
    def _zero():
        pltpu.sync_copy(zeros_hbm, acc_sh)

    plsc.subcore_barrier()

    @pl.loop(0, _ROUNDS)
    def _round(i):
        blk = i * _NW + wid

        @pl.when(blk < _NBLK)
        def _do_block():
            base = blk * _B
            pltpu.sync_copy(ei0_hbm.at[pl.ds(base, _B)], ei_v)
            pltpu.sync_copy(edge_hbm.at[pl.ds(base, _B)], rows_v)
            for j in range(_B // 16):
                idx16 = ei_v[pl.ds(j * 16, 16)]
                gid_v[pl.ds(j * 16, 16)] = plsc.load_gather(batch_v, [idx16])
            pltpu.sync_copy(rows_v, acc_sh.at[gid_v], add=True)

    plsc.subcore_barrier()

    @pl.when(s == 0)
    def _flush():
        pltpu.sync_copy(acc_sh, out_hbm.at[c])


def _dense_body(p_ref, w1_ref, w2_ref, w3_ref, gamma_ref, beta_ref,
                mean_ref, var_ref, out_ref):
    g = p_ref[0] + p_ref[1]
    cdims = (((1,), (1,)), ((), ()))
    h = lax.dot_general(g, w1_ref[...], cdims,
                        preferred_element_type=jnp.float32)
    h = lax.dot_general(h, w2_ref[...], cdims,
                        preferred_element_type=jnp.float32)
    h = (h - mean_ref[...]) * lax.rsqrt(var_ref[...] + _EPS)
    h = h * gamma_ref[...] + beta_ref[...]
    h = jax.nn.gelu(h)
    out_ref[...] = lax.dot_general(h, w3_ref[...], cdims,
                                   preferred_element_type=jnp.float32)


def kernel(edge, batch, edge_index, W1, W2, W3, gamma, beta,
           running_mean, running_var):
    ei0 = edge_index[0]
    zeros = jnp.zeros((_NG, _H), jnp.float32)

    sc_call = pl.kernel(
        _sc_agg_body,
        out_type=jax.ShapeDtypeStruct((_NC, _NG, _H), jnp.float32),
        mesh=plsc.VectorSubcoreMesh(core_axis_name="c", subcore_axis_name="s"),
        scratch_types=[
            pltpu.VMEM((_N,), jnp.int32),
            pltpu.VMEM((_B,), jnp.int32),
            pltpu.VMEM((_B,), jnp.int32),
            pltpu.VMEM((_B, _H), jnp.float32),
            pltpu.VMEM_SHARED((_NG, _H), jnp.float32),
        ],
        compiler_params=pltpu.CompilerParams(needs_layout_passes=False),
    )
    partials = sc_call(edge, ei0, batch, zeros)

    out = pl.pallas_call(
        _dense_body,
        out_shape=jax.ShapeDtypeStruct((_NG, _H), jnp.float32),
    )(partials, W1, W2, W3,
      gamma.reshape(1, _H), beta.reshape(1, _H),
      running_mean.reshape(1, _H), running_var.reshape(1, _H))
    return out


# speedup vs baseline: 19.2147x; 1.6509x over previous
import jax
import jax.numpy as jnp
from jax import lax
from jax.experimental import pallas as pl
from jax.experimental.pallas import tpu as pltpu
from jax.experimental.pallas import tpu_sc as plsc

_E = 320000
_N = 10000
_H = 128
_NG = 512
_EPS = 1e-5

_NC = 2
_NS = 16
_NW = _NC * _NS
_EPW = _E // _NW
_B = 80
_NBLK = _EPW // _B


def _sc_agg_body(edge_hbm, ei0_hbm, batch_hbm, zeros_hbm, out_hbm,
                 batch_v, ei_v, gid_v, rows0, rows1, sem0, sem1, acc_sh):
    c = lax.axis_index("c")
    s = lax.axis_index("s")
    wid = s * _NC + c
    ebase = wid * _EPW
    rows = (rows0, rows1)
    sems = (sem0, sem1)

    pltpu.sync_copy(batch_hbm, batch_v)
    pltpu.sync_copy(ei0_hbm.at[pl.ds(ebase, _EPW)], ei_v)

    @pl.when(s == 0)
    def _zero():
        pltpu.sync_copy(zeros_hbm, acc_sh)

    plsc.subcore_barrier()

    def _start(b, par):
        pltpu.async_copy(edge_hbm.at[pl.ds(ebase + b * _B, _B)],
                         rows[par], sems[par])

    def _consume(b, par):
        pltpu.make_async_copy(edge_hbm.at[pl.ds(ebase + b * _B, _B)],
                              rows[par], sems[par]).wait()
        boff = b * _B
        for j in range(_B // 16):
            idx16 = ei_v[pl.ds(boff + j * 16, 16)]
            gid_v[pl.ds(j * 16, 16)] = plsc.load_gather(batch_v, [idx16])
        pltpu.sync_copy(rows[par], acc_sh.at[gid_v], add=True)

    _start(0, 0)

    @pl.loop(0, _NBLK - 1, step=2)
    def _round(b):
        for par in range(2):
            _start(b + par + 1, (par + 1) % 2)
            _consume(b + par, par)

    _consume(_NBLK - 1, (_NBLK - 1) % 2)

    plsc.subcore_barrier()

    @pl.when(s == 0)
    def _flush():
        pltpu.sync_copy(acc_sh, out_hbm.at[c])


def _dense_body(p_ref, w1_ref, w2_ref, w3_ref, gamma_ref, beta_ref,
                mean_ref, var_ref, out_ref):
    g = p_ref[0] + p_ref[1]
    cdims = (((1,), (1,)), ((), ()))
    h = lax.dot_general(g, w1_ref[...], cdims,
                        preferred_element_type=jnp.float32)
    h = lax.dot_general(h, w2_ref[...], cdims,
                        preferred_element_type=jnp.float32)
    h = (h - mean_ref[...]) * lax.rsqrt(var_ref[...] + _EPS)
    h = h * gamma_ref[...] + beta_ref[...]
    h = jax.nn.gelu(h)
    out_ref[...] = lax.dot_general(h, w3_ref[...], cdims,
                                   preferred_element_type=jnp.float32)


def kernel(edge, batch, edge_index, W1, W2, W3, gamma, beta,
           running_mean, running_var):
    ei0 = edge_index[0]
    zeros = jnp.zeros((_NG, _H), jnp.float32)

    sc_call = pl.kernel(
        _sc_agg_body,
        out_type=jax.ShapeDtypeStruct((_NC, _NG, _H), jnp.float32),
        mesh=plsc.VectorSubcoreMesh(core_axis_name="c", subcore_axis_name="s"),
        scratch_types=[
            pltpu.VMEM((_N,), jnp.int32),
            pltpu.VMEM((_EPW,), jnp.int32),
            pltpu.VMEM((_B,), jnp.int32),
            pltpu.VMEM((_B, _H), jnp.float32),
            pltpu.VMEM((_B, _H), jnp.float32),
            pltpu.SemaphoreType.DMA,
            pltpu.SemaphoreType.DMA,
            pltpu.VMEM_SHARED((_NG, _H), jnp.float32),
        ],
        compiler_params=pltpu.CompilerParams(needs_layout_passes=False),
    )
    partials = sc_call(edge, ei0, batch, zeros)

    out = pl.pallas_call(
        _dense_body,
        out_shape=jax.ShapeDtypeStruct((_NG, _H), jnp.float32),
    )(partials, W1, W2, W3,
      gamma.reshape(1, _H), beta.reshape(1, _H),
      running_mean.reshape(1, _H), running_var.reshape(1, _H))
    return out


# speedup vs baseline: 19.5414x; 1.0170x over previous
import jax
import jax.numpy as jnp
from jax import lax
from jax.experimental import pallas as pl
from jax.experimental.pallas import tpu as pltpu
from jax.experimental.pallas import tpu_sc as plsc

_E = 320000
_N = 10000
_H = 128
_NG = 512
_EPS = 1e-5

_NC = 2
_NS = 16
_NW = _NC * _NS
_EPW = _E // _NW
_B = 80
_NBLK = _EPW // _B


_DEPTH = 4
_LOOP_HI = ((_NBLK + _DEPTH - 1) // _DEPTH) * _DEPTH


def _sc_agg_body(edge_hbm, ei0_hbm, batch_hbm, zeros_hbm, out_hbm,
                 batch_v, ei_v,
                 rows0, rows1, rows2, rows3,
                 gid0, gid1, gid2, gid3,
                 dsem0, dsem1, dsem2, dsem3,
                 ssem0, ssem1, ssem2, ssem3,
                 acc_sh):
    c = lax.axis_index("c")
    s = lax.axis_index("s")
    wid = s * _NC + c
    ebase = wid * _EPW
    rows = (rows0, rows1, rows2, rows3)
    gids = (gid0, gid1, gid2, gid3)
    dsems = (dsem0, dsem1, dsem2, dsem3)
    ssems = (ssem0, ssem1, ssem2, ssem3)

    pltpu.sync_copy(batch_hbm, batch_v)
    pltpu.sync_copy(ei0_hbm.at[pl.ds(ebase, _EPW)], ei_v)

    @pl.when(s == 0)
    def _zero():
        pltpu.sync_copy(zeros_hbm, acc_sh)

    plsc.subcore_barrier()

    def _start_in(b, par):
        pltpu.async_copy(edge_hbm.at[pl.ds(ebase + b * _B, _B)],
                         rows[par], dsems[par])

    def _wait_in(b, par):
        pltpu.make_async_copy(edge_hbm.at[pl.ds(ebase + b * _B, _B)],
                              rows[par], dsems[par]).wait()

    def _wait_scat(par):
        pltpu.make_async_copy(rows[par], acc_sh.at[gids[par]],
                              ssems[par]).wait()

    _start_in(0, 0)

    @pl.loop(0, _LOOP_HI, step=_DEPTH)
    def _round(b0):
        for par in range(_DEPTH):
            b = b0 + par

            @pl.when(jnp.logical_and(b >= _DEPTH - 1, b + 1 < _NBLK))
            def _pf():
                _wait_scat((par + 1) % _DEPTH)
                _start_in(b + 1, (par + 1) % _DEPTH)

            @pl.when(b + 1 < _DEPTH)
            def _pf0():
                _start_in(b + 1, (par + 1) % _DEPTH)

            @pl.when(b < _NBLK)
            def _do():
                _wait_in(b, par)
                boff = b * _B
                for j in range(_B // 16):
                    idx16 = ei_v[pl.ds(boff + j * 16, 16)]
                    gids[par][pl.ds(j * 16, 16)] = \
                        plsc.load_gather(batch_v, [idx16])
                pltpu.async_copy(rows[par], acc_sh.at[gids[par]],
                                 ssems[par], add=True)


    for b in range(_NBLK - _DEPTH, _NBLK):
        _wait_scat(b % _DEPTH)

    plsc.subcore_barrier()

    @pl.when(s == 0)
    def _flush():
        pltpu.sync_copy(acc_sh, out_hbm.at[c])


def _dense_body(p_ref, w1_ref, w2_ref, w3_ref, gamma_ref, beta_ref,
                mean_ref, var_ref, out_ref):
    g = p_ref[0] + p_ref[1]
    cdims = (((1,), (1,)), ((), ()))
    h = lax.dot_general(g, w1_ref[...], cdims,
                        preferred_element_type=jnp.float32)
    h = lax.dot_general(h, w2_ref[...], cdims,
                        preferred_element_type=jnp.float32)
    h = (h - mean_ref[...]) * lax.rsqrt(var_ref[...] + _EPS)
    h = h * gamma_ref[...] + beta_ref[...]
    h = jax.nn.gelu(h)
    out_ref[...] = lax.dot_general(h, w3_ref[...], cdims,
                                   preferred_element_type=jnp.float32)


def kernel(edge, batch, edge_index, W1, W2, W3, gamma, beta,
           running_mean, running_var):
    ei0 = edge_index[0]
    zeros = jnp.zeros((_NG, _H), jnp.float32)

    sc_call = pl.kernel(
        _sc_agg_body,
        out_type=jax.ShapeDtypeStruct((_NC, _NG, _H), jnp.float32),
        mesh=plsc.VectorSubcoreMesh(core_axis_name="c", subcore_axis_name="s"),
        scratch_types=(
            [pltpu.VMEM((_N,), jnp.int32),
             pltpu.VMEM((_EPW,), jnp.int32)]
            + [pltpu.VMEM((_B, _H), jnp.float32)] * _DEPTH
            + [pltpu.VMEM((_B,), jnp.int32)] * _DEPTH
            + [pltpu.SemaphoreType.DMA] * (2 * _DEPTH)
            + [pltpu.VMEM_SHARED((_NG, _H), jnp.float32)]
        ),
        compiler_params=pltpu.CompilerParams(needs_layout_passes=False),
    )
    partials = sc_call(edge, ei0, batch, zeros)

    out = pl.pallas_call(
        _dense_body,
        out_shape=jax.ShapeDtypeStruct((_NG, _H), jnp.float32),
    )(partials, W1, W2, W3,
      gamma.reshape(1, _H), beta.reshape(1, _H),
      running_mean.reshape(1, _H), running_var.reshape(1, _H))
    return out
